# baseline (speedup 1.0000x reference)
import jax
import jax.numpy as jnp
from jax.experimental import pallas as pl


def _add_kernel(patch_ref, pos_ref, out_ref):
    out_ref[...] = patch_ref[...] + pos_ref[...]


_BB = 8


def _tc_add(patch, pos_table):
    B, N, D = patch.shape
    bb = min(_BB, B)
    return pl.pallas_call(
        _add_kernel,
        grid=(B // bb,),
        in_specs=[
            pl.BlockSpec((bb, N, D), lambda b: (b, 0, 0)),
            pl.BlockSpec((N, D), lambda b: (0, 0)),
        ],
        out_specs=pl.BlockSpec((bb, N, D), lambda b: (b, 0, 0)),
        out_shape=jax.ShapeDtypeStruct((B, N, D), patch.dtype),
    )(patch, pos_table)


def kernel(patch, pos_table):
    B, N, D = patch.shape
    split = 8
    lo = _tc_add(patch[:split], pos_table)
    hi = _tc_add(patch[split:], pos_table)
    return jnp.concatenate([lo, hi], axis=0)


# speedup vs baseline: 3.0905x; 3.0905x over previous
import jax
import jax.numpy as jnp
from jax.experimental import pallas as pl
from jax.experimental.pallas import tpu as pltpu


def _add_kernel(patch_ref, pos_ref, out_ref):
    out_ref[...] = patch_ref[...] + pos_ref[...][None]


_BB = 8


def kernel(patch, pos_table):
    B, N, D = patch.shape
    return pl.pallas_call(
        _add_kernel,
        grid=(B // _BB,),
        in_specs=[
            pl.BlockSpec((_BB, N, D), lambda b: (b, 0, 0)),
            pl.BlockSpec(memory_space=pltpu.VMEM),
        ],
        out_specs=pl.BlockSpec((_BB, N, D), lambda b: (b, 0, 0)),
        out_shape=jax.ShapeDtypeStruct((B, N, D), patch.dtype),
    )(patch, pos_table)
